# baseline (speedup 1.0000x reference)
import jax
import jax.numpy as jnp
from jax.experimental import pallas as pl
from jax.experimental.pallas import tpu as pltpu

_NUM_UB = 8


def _topk_mask(w):
    num_charts = w.shape[1]
    lane = jax.lax.broadcasted_iota(jnp.int32, w.shape, 1)
    m0 = jnp.max(w, axis=1, keepdims=True)
    e0 = jnp.min(jnp.where(w == m0, lane, num_charts), axis=1, keepdims=True)
    oh0 = lane == e0
    w_rest = jnp.where(oh0, -jnp.inf, w)
    m1 = jnp.max(w_rest, axis=1, keepdims=True)
    e1 = jnp.min(jnp.where(w_rest == m1, lane, num_charts), axis=1, keepdims=True)
    oh1 = lane == e1
    denom = jnp.maximum(m0 + m1, 1e-8)
    return (jnp.where(oh0, m0, 0.0) + jnp.where(oh1, m1, 0.0)) / denom


def _moe_body(w_ref, qa_ref, qs_ref, wblk_ref, o_ref, qb_ref, wm_ref):
    uc = pl.program_id(0)
    k = pl.program_id(1)
    ub = wblk_ref.shape[0]

    @pl.when(jnp.logical_and(uc == 0, k == 0))
    def _():
        qb_ref[...] = qa_ref[...].astype(jnp.bfloat16)
        wm_ref[...] = _topk_mask(w_ref[...])

    @pl.when(k == 0)
    def _():
        s = jnp.sum(wm_ref[...], axis=1, keepdims=True)
        o_ref[...] = (-s) * qs_ref[...]

    wb = wblk_ref[...]
    ri = jax.lax.broadcasted_iota(jnp.int32, wb.shape, 0)
    ci = jax.lax.broadcasted_iota(jnp.int32, wb.shape, 1)
    eb = (wb + jnp.where(ci == ri + uc * ub, 1.0, 0.0)).astype(jnp.bfloat16)

    y = jax.lax.dot_general(
        qb_ref[...], eb,
        dimension_numbers=(((1,), (1,)), ((), ())),
        preferred_element_type=jnp.float32,
    )

    lane = jax.lax.broadcasted_iota(jnp.int32, wm_ref.shape, 1)
    wcol = jnp.sum(jnp.where(lane == k, wm_ref[...], 0.0), axis=1, keepdims=True)
    o_ref[...] += wcol * y


@jax.jit
def kernel(q, weights, W_stack):
    b, d = q.shape
    c = W_stack.shape[0]
    w_flat = W_stack.reshape(c * d, d)
    ub = d // _NUM_UB

    return pl.pallas_call(
        _moe_body,
        grid=(_NUM_UB, c),
        in_specs=[
            pl.BlockSpec((b, c), lambda uc, k: (0, 0)),
            pl.BlockSpec((b, d), lambda uc, k: (0, 0)),
            pl.BlockSpec((b, ub), lambda uc, k: (0, uc)),
            pl.BlockSpec((ub, d), lambda uc, k: (k * _NUM_UB + uc, 0)),
        ],
        out_specs=pl.BlockSpec((b, ub), lambda uc, k: (0, uc)),
        out_shape=jax.ShapeDtypeStruct((b, d), jnp.float32),
        scratch_shapes=[
            pltpu.VMEM((b, d), jnp.bfloat16),
            pltpu.VMEM((b, c), jnp.float32),
        ],
        compiler_params=pltpu.CompilerParams(
            dimension_semantics=("arbitrary", "arbitrary"),
        ),
    )(weights, q, q, w_flat)


# speedup vs baseline: 1.2431x; 1.2431x over previous
import jax
import jax.numpy as jnp
from jax.experimental import pallas as pl
from jax.experimental.pallas import tpu as pltpu

_TOKEN_BLOCK = 1024


def _moe_body(w_ref, q_ref, r_ref, o_ref):
    w = w_ref[...]
    num_charts = w.shape[1]
    lane = jax.lax.broadcasted_iota(jnp.int32, w.shape, 1)

    m0 = jnp.max(w, axis=1, keepdims=True)
    e0 = jnp.min(jnp.where(w == m0, lane, num_charts), axis=1, keepdims=True)
    oh0 = lane == e0
    w_rest = jnp.where(oh0, -jnp.inf, w)
    m1 = jnp.max(w_rest, axis=1, keepdims=True)
    e1 = jnp.min(jnp.where(w_rest == m1, lane, num_charts), axis=1, keepdims=True)
    oh1 = lane == e1

    denom = jnp.maximum(m0 + m1, 1e-8)
    wm = (jnp.where(oh0, m0, 0.0) + jnp.where(oh1, m1, 0.0)) / denom
    s = (m0 + m1) / denom

    qf = q_ref[...]
    qb = qf.astype(jnp.bfloat16)
    ys = [
        jax.lax.dot_general(
            qb, r_ref[k],
            dimension_numbers=(((1,), (0,)), ((), ())),
            preferred_element_type=jnp.float32,
        )
        for k in range(num_charts)
    ]
    acc = (-s) * qf
    for k in range(num_charts):
        acc = acc + wm[:, k:k + 1] * ys[k]
    o_ref[...] = acc


@jax.jit
def kernel(q, weights, W_stack):
    b, d = q.shape
    c = W_stack.shape[0]
    resid_t = (W_stack.transpose(0, 2, 1)
               + jnp.eye(d, dtype=W_stack.dtype)).astype(jnp.bfloat16)

    t = _TOKEN_BLOCK
    grid = (b // t,)
    return pl.pallas_call(
        _moe_body,
        grid=grid,
        in_specs=[
            pl.BlockSpec((t, c), lambda i: (i, 0)),
            pl.BlockSpec((t, d), lambda i: (i, 0)),
            pl.BlockSpec((c, d, d), lambda i: (0, 0, 0)),
        ],
        out_specs=pl.BlockSpec((t, d), lambda i: (i, 0)),
        out_shape=jax.ShapeDtypeStruct((b, d), jnp.float32),
        compiler_params=pltpu.CompilerParams(
            dimension_semantics=("arbitrary",),
        ),
    )(weights, q, resid_t)


# speedup vs baseline: 1.7197x; 1.3834x over previous
import jax
import jax.numpy as jnp
from jax.experimental import pallas as pl
from jax.experimental.pallas import tpu as pltpu

_TOKEN_BLOCK = 1024


def _moe_body(w_ref, q_ref, r_ref, o_ref):
    w = w_ref[...]
    num_charts = w.shape[1]
    lane = jax.lax.broadcasted_iota(jnp.int32, w.shape, 1)

    m0 = jnp.max(w, axis=1, keepdims=True)
    e0 = jnp.min(jnp.where(w == m0, lane, num_charts), axis=1, keepdims=True)
    oh0 = lane == e0
    w_rest = jnp.where(oh0, -jnp.inf, w)
    m1 = jnp.max(w_rest, axis=1, keepdims=True)
    e1 = jnp.min(jnp.where(w_rest == m1, lane, num_charts), axis=1, keepdims=True)
    oh1 = lane == e1

    denom = jnp.maximum(m0 + m1, 1e-8)
    wm = (jnp.where(oh0, m0, 0.0) + jnp.where(oh1, m1, 0.0)) / denom
    s = (m0 + m1) / denom

    qf = q_ref[...]
    qb = qf.astype(jnp.bfloat16)
    ys = [
        jax.lax.dot_general(
            qb, r_ref[k],
            dimension_numbers=(((1,), (1,)), ((), ())),
            preferred_element_type=jnp.float32,
        )
        for k in range(num_charts)
    ]
    acc = (-s) * qf
    for k in range(num_charts):
        acc = acc + wm[:, k:k + 1] * ys[k]
    o_ref[...] = acc


@jax.jit
def kernel(q, weights, W_stack):
    b, d = q.shape
    c = W_stack.shape[0]
    resid_t = (W_stack + jnp.eye(d, dtype=W_stack.dtype)).astype(jnp.bfloat16)

    t = _TOKEN_BLOCK
    grid = (b // t,)
    return pl.pallas_call(
        _moe_body,
        grid=grid,
        in_specs=[
            pl.BlockSpec((t, c), lambda i: (i, 0)),
            pl.BlockSpec((t, d), lambda i: (i, 0)),
            pl.BlockSpec((c, d, d), lambda i: (0, 0, 0)),
        ],
        out_specs=pl.BlockSpec((t, d), lambda i: (i, 0)),
        out_shape=jax.ShapeDtypeStruct((b, d), jnp.float32),
        compiler_params=pltpu.CompilerParams(
            dimension_semantics=("arbitrary",),
        ),
    )(weights, q, resid_t)
